# baseline (speedup 1.0000x reference)
import jax
import jax.numpy as jnp
from jax import lax
from jax.experimental import pallas as pl
from jax.experimental.pallas import tpu as pltpu

N_LAYER = 6
B, S, V, D, H = 2, 1024, 32000, 768, 12
T = S + 1
VBLK = 1280
VSUB = VBLK // 2
NV = V // VBLK
GBLK = 128

_CONTRACT_LAST = (((1,), (1,)), ((), ()))


def _gather_body(idx_ref, we_ref, out_ref, sem):
    g = pl.program_id(0)
    base = g * GBLK
    for mi in range(GBLK):
        row = idx_ref[base + mi]
        pltpu.make_async_copy(
            we_ref.at[pl.ds(row, 1), :], out_ref.at[0, pl.ds(mi, 1), :], sem
        ).start()
    for mi in range(GBLK):
        pltpu.make_async_copy(
            we_ref.at[pl.ds(0, 1), :], out_ref.at[0, pl.ds(mi, 1), :], sem
        ).wait()


def _k_body(p_ref, wk_ref, wq_ref, k_ref):
    xi = jnp.dot(p_ref[:S], wk_ref[0], preferred_element_type=jnp.float32)
    xj = jnp.dot(p_ref[...], wq_ref[0], preferred_element_type=jnp.float32)
    k_ref[0] = lax.dot_general(xj, xi, _CONTRACT_LAST,
                               preferred_element_type=jnp.float32)


def _vocab_body(f_ref, wea_ref, web_ref, e_ref, out_ref, num_ref, den_ref):
    j = pl.program_id(1)

    @pl.when(j == 0)
    def _():
        num_ref[...] = jnp.zeros_like(num_ref)
        den_ref[...] = jnp.zeros_like(den_ref)

    fs = f_ref[0, :S]

    def chunk(we_ref):
        we = we_ref[...]
        ex = jnp.exp(lax.dot_general(fs, we, _CONTRACT_LAST,
                                     preferred_element_type=jnp.float32))
        num_ref[...] += jnp.dot(ex, we, preferred_element_type=jnp.float32)
        den_ref[...] += jnp.sum(ex, axis=1, keepdims=True)

    @pl.when(2 * j + 1 < NV)
    def _():
        chunk(wea_ref)
        chunk(web_ref)

    @pl.when(2 * j + 1 >= NV)
    def _():
        chunk(wea_ref)
        den = den_ref[:, :1] + 1e-8
        out_ref[0] = e_ref[0] - num_ref[...] / den


def _head_body(diff_ref, k_ref, wv_ref, f_ref, a_ref, b_ref, out_ref, acc_ref):
    j = pl.program_id(1)

    @pl.when(j == 0)
    def _():
        db = jnp.sum(diff_ref[0], axis=0, keepdims=True) * b_ref[0, 0]
        acc_ref[...] = jnp.broadcast_to(db, acc_ref.shape)

    diff = diff_ref[0]
    for u in range(2):
        tmp = jnp.dot(diff, wv_ref[u],
                      preferred_element_type=jnp.float32) * a_ref[0, 2 * j + u]
        acc_ref[...] += jnp.dot(k_ref[u], tmp,
                                preferred_element_type=jnp.float32)

    @pl.when(j == H // 2 - 1)
    def _():
        out_ref[0] = f_ref[0] + acc_ref[...] * (1.0 / S)


def _logits_body(fl_ref, we_ref, out_ref):
    out_ref[...] = lax.dot_general(fl_ref[...], we_ref[...], _CONTRACT_LAST,
                                   preferred_element_type=jnp.float32)


def kernel(idx, W_e, W_p, W_k, W_q, W_v, A_LR, B_LR):
    f32 = jnp.float32
    p = W_p[:T]
    A2 = A_LR.reshape(1, H).astype(f32)
    B2 = B_LR.reshape(1, 1).astype(f32)

    e = pl.pallas_call(
        _gather_body,
        out_shape=jax.ShapeDtypeStruct((B, S, D), f32),
        grid_spec=pltpu.PrefetchScalarGridSpec(
            num_scalar_prefetch=1,
            grid=(B * S // GBLK,),
            in_specs=[pl.BlockSpec(memory_space=pl.ANY)],
            out_specs=pl.BlockSpec(
                (1, GBLK, D), lambda g, i: (g // (S // GBLK), g % (S // GBLK), 0)),
            scratch_shapes=[pltpu.SemaphoreType.DMA],
        ),
        compiler_params=pltpu.CompilerParams(
            dimension_semantics=("arbitrary",)),
        name="pgd_gather",
    )(idx.reshape(-1).astype(jnp.int32), W_e)

    K = pl.pallas_call(
        _k_body,
        out_shape=jax.ShapeDtypeStruct((H, T, S), f32),
        grid=(H,),
        in_specs=[
            pl.BlockSpec((T, D), lambda h: (0, 0)),
            pl.BlockSpec((1, D, D), lambda h: (h, 0, 0)),
            pl.BlockSpec((1, D, D), lambda h: (h, 0, 0)),
        ],
        out_specs=pl.BlockSpec((1, T, S), lambda h: (h, 0, 0)),
        compiler_params=pltpu.CompilerParams(
            dimension_semantics=("arbitrary",),
            vmem_limit_bytes=56 * 1024 * 1024),
        name="pgd_kmat",
    )(p, W_k, W_q)

    vocab_call = pl.pallas_call(
        _vocab_body,
        out_shape=jax.ShapeDtypeStruct((B, S, D), f32),
        grid=(B, (NV + 1) // 2),
        in_specs=[
            pl.BlockSpec((1, T, D), lambda b, j: (b, 0, 0)),
            pl.BlockSpec((VBLK, D), lambda b, j: (2 * j, 0)),
            pl.BlockSpec((VBLK, D),
                         lambda b, j: (jnp.minimum(2 * j + 1, NV - 1), 0)),
            pl.BlockSpec((1, S, D), lambda b, j: (b, 0, 0)),
        ],
        out_specs=pl.BlockSpec((1, S, D), lambda b, j: (b, 0, 0)),
        scratch_shapes=[
            pltpu.VMEM((S, D), f32),
            pltpu.VMEM((S, 128), f32),
        ],
        compiler_params=pltpu.CompilerParams(
            dimension_semantics=("parallel", "arbitrary"),
            vmem_limit_bytes=56 * 1024 * 1024),
        name="pgd_vocab",
    )

    head_call = pl.pallas_call(
        _head_body,
        out_shape=jax.ShapeDtypeStruct((B, T, D), f32),
        grid=(B, H // 2),
        in_specs=[
            pl.BlockSpec((1, S, D), lambda b, j: (b, 0, 0)),
            pl.BlockSpec((2, T, S), lambda b, j: (j, 0, 0)),
            pl.BlockSpec((2, D, D), lambda b, j: (j, 0, 0)),
            pl.BlockSpec((1, T, D), lambda b, j: (b, 0, 0)),
            pl.BlockSpec(memory_space=pltpu.SMEM),
            pl.BlockSpec(memory_space=pltpu.SMEM),
        ],
        out_specs=pl.BlockSpec((1, T, D), lambda b, j: (b, 0, 0)),
        scratch_shapes=[pltpu.VMEM((T, D), f32)],
        compiler_params=pltpu.CompilerParams(
            dimension_semantics=("parallel", "arbitrary"),
            vmem_limit_bytes=56 * 1024 * 1024),
        name="pgd_head",
    )

    f = jnp.zeros((B, T, D), f32)
    for _ in range(N_LAYER):
        diff = vocab_call(f, W_e, W_e, e)
        f = head_call(diff, K, W_v, f, A2, B2)

    fl = jnp.pad(f[:, S], ((0, 8 - B), (0, 0)))
    LBLK = 3200
    lg = pl.pallas_call(
        _logits_body,
        out_shape=jax.ShapeDtypeStruct((8, V), f32),
        grid=(V // LBLK,),
        in_specs=[
            pl.BlockSpec((8, D), lambda v: (0, 0)),
            pl.BlockSpec((LBLK, D), lambda v: (v, 0)),
        ],
        out_specs=pl.BlockSpec((8, LBLK), lambda v: (0, v)),
        compiler_params=pltpu.CompilerParams(
            dimension_semantics=("arbitrary",),
            vmem_limit_bytes=56 * 1024 * 1024),
        name="pgd_logits",
    )(fl, W_e)
    return lg[:B, :V]


# speedup vs baseline: 1.0253x; 1.0253x over previous
import jax
import jax.numpy as jnp
from jax import lax
from jax.experimental import pallas as pl
from jax.experimental.pallas import tpu as pltpu

N_LAYER = 6
B, S, V, D, H = 2, 1024, 32000, 768, 12
T = S + 1
VBLK = 1280
VSUB = VBLK // 2
NV = V // VBLK
GBLK = 128

_CONTRACT_LAST = (((1,), (1,)), ((), ()))


def _gather_body(idx_ref, we_ref, out_ref, sem):
    g = pl.program_id(0)
    base = g * GBLK
    for mi in range(GBLK):
        row = idx_ref[base + mi]
        pltpu.make_async_copy(
            we_ref.at[pl.ds(row, 1), :], out_ref.at[0, pl.ds(mi, 1), :], sem
        ).start()
    for mi in range(GBLK):
        pltpu.make_async_copy(
            we_ref.at[pl.ds(0, 1), :], out_ref.at[0, pl.ds(mi, 1), :], sem
        ).wait()


def _k_body(p_ref, wk_ref, wq_ref, k_ref):
    for u in range(2):
        xi = jnp.dot(p_ref[:S], wk_ref[u], preferred_element_type=jnp.float32)
        xj = jnp.dot(p_ref[...], wq_ref[u], preferred_element_type=jnp.float32)
        k_ref[u] = lax.dot_general(xj, xi, _CONTRACT_LAST,
                                   preferred_element_type=jnp.float32)


def _vocab_body(f_ref, wea_ref, e_ref, out_ref, num_ref, den_ref):
    j = pl.program_id(1)

    @pl.when(j == 0)
    def _():
        num_ref[...] = jnp.zeros_like(num_ref)
        den_ref[...] = jnp.zeros_like(den_ref)

    fs = f_ref[0, :S]

    def chunk(we_ref):
        we = we_ref[...]
        ex = jnp.exp(lax.dot_general(fs, we, _CONTRACT_LAST,
                                     preferred_element_type=jnp.float32))
        num_ref[...] += jnp.dot(ex, we, preferred_element_type=jnp.float32)
        den_ref[...] += jnp.sum(ex, axis=1, keepdims=True)

    chunk(wea_ref)

    @pl.when(j == NV - 1)
    def _():
        den = den_ref[:, :1] + 1e-8
        out_ref[0] = e_ref[0] - num_ref[...] / den


def _head_body(diff_ref, k_ref, wv_ref, f_ref, a_ref, b_ref, out_ref, acc_ref):
    j = pl.program_id(1)

    @pl.when(j == 0)
    def _():
        db = jnp.sum(diff_ref[0], axis=0, keepdims=True) * b_ref[0, 0]
        acc_ref[...] = jnp.broadcast_to(db, acc_ref.shape)

    tmp = jnp.dot(diff_ref[0], wv_ref[0],
                  preferred_element_type=jnp.float32) * a_ref[0, j]
    acc_ref[...] += jnp.dot(k_ref[0], tmp, preferred_element_type=jnp.float32)

    @pl.when(j == H - 1)
    def _():
        out_ref[0] = f_ref[0] + acc_ref[...] * (1.0 / S)


def _logits_body(fl_ref, we_ref, out_ref):
    out_ref[...] = lax.dot_general(fl_ref[...], we_ref[...], _CONTRACT_LAST,
                                   preferred_element_type=jnp.float32)


def kernel(idx, W_e, W_p, W_k, W_q, W_v, A_LR, B_LR):
    f32 = jnp.float32
    p = W_p[:T]
    A2 = A_LR.reshape(1, H).astype(f32)
    B2 = B_LR.reshape(1, 1).astype(f32)

    e = pl.pallas_call(
        _gather_body,
        out_shape=jax.ShapeDtypeStruct((B, S, D), f32),
        grid_spec=pltpu.PrefetchScalarGridSpec(
            num_scalar_prefetch=1,
            grid=(B * S // GBLK,),
            in_specs=[pl.BlockSpec(memory_space=pl.ANY)],
            out_specs=pl.BlockSpec(
                (1, GBLK, D), lambda g, i: (g // (S // GBLK), g % (S // GBLK), 0)),
            scratch_shapes=[pltpu.SemaphoreType.DMA],
        ),
        compiler_params=pltpu.CompilerParams(
            dimension_semantics=("arbitrary",)),
        name="pgd_gather",
    )(idx.reshape(-1).astype(jnp.int32), W_e)

    K = pl.pallas_call(
        _k_body,
        out_shape=jax.ShapeDtypeStruct((H, T, S), f32),
        grid=(H // 2,),
        in_specs=[
            pl.BlockSpec((T, D), lambda h: (0, 0)),
            pl.BlockSpec((2, D, D), lambda h: (h, 0, 0)),
            pl.BlockSpec((2, D, D), lambda h: (h, 0, 0)),
        ],
        out_specs=pl.BlockSpec((2, T, S), lambda h: (h, 0, 0)),
        compiler_params=pltpu.CompilerParams(
            dimension_semantics=("arbitrary",),
            vmem_limit_bytes=56 * 1024 * 1024),
        name="pgd_kmat",
    )(p, W_k, W_q)

    vocab_call = pl.pallas_call(
        _vocab_body,
        out_shape=jax.ShapeDtypeStruct((B, S, D), f32),
        grid=(B, NV),
        in_specs=[
            pl.BlockSpec((1, T, D), lambda b, j: (b, 0, 0)),
            pl.BlockSpec((VBLK, D), lambda b, j: (j, 0)),
            pl.BlockSpec((1, S, D), lambda b, j: (b, 0, 0)),
        ],
        out_specs=pl.BlockSpec((1, S, D), lambda b, j: (b, 0, 0)),
        scratch_shapes=[
            pltpu.VMEM((S, D), f32),
            pltpu.VMEM((S, 128), f32),
        ],
        compiler_params=pltpu.CompilerParams(
            dimension_semantics=("parallel", "arbitrary"),
            vmem_limit_bytes=56 * 1024 * 1024),
        name="pgd_vocab",
    )

    head_call = pl.pallas_call(
        _head_body,
        out_shape=jax.ShapeDtypeStruct((B, T, D), f32),
        grid=(B, H),
        in_specs=[
            pl.BlockSpec((1, S, D), lambda b, j: (b, 0, 0)),
            pl.BlockSpec((1, T, S), lambda b, j: (j, 0, 0)),
            pl.BlockSpec((1, D, D), lambda b, j: (j, 0, 0)),
            pl.BlockSpec((1, T, D), lambda b, j: (b, 0, 0)),
            pl.BlockSpec(memory_space=pltpu.SMEM),
            pl.BlockSpec(memory_space=pltpu.SMEM),
        ],
        out_specs=pl.BlockSpec((1, T, D), lambda b, j: (b, 0, 0)),
        scratch_shapes=[pltpu.VMEM((T, D), f32)],
        compiler_params=pltpu.CompilerParams(
            dimension_semantics=("parallel", "arbitrary"),
            vmem_limit_bytes=56 * 1024 * 1024),
        name="pgd_head",
    )

    f = jnp.zeros((B, T, D), f32)
    for _ in range(N_LAYER):
        diff = vocab_call(f, W_e, e)
        f = head_call(diff, K, W_v, f, A2, B2)

    fl = jnp.pad(f[:, S], ((0, 8 - B), (0, 0)))
    LBLK = 3200
    lg = pl.pallas_call(
        _logits_body,
        out_shape=jax.ShapeDtypeStruct((8, V), f32),
        grid=(V // LBLK,),
        in_specs=[
            pl.BlockSpec((8, D), lambda v: (0, 0)),
            pl.BlockSpec((LBLK, D), lambda v: (v, 0)),
        ],
        out_specs=pl.BlockSpec((8, LBLK), lambda v: (0, v)),
        compiler_params=pltpu.CompilerParams(
            dimension_semantics=("arbitrary",),
            vmem_limit_bytes=56 * 1024 * 1024),
        name="pgd_logits",
    )(fl, W_e)
    return lg[:B, :V]
